# Initial kernel scaffold; baseline (speedup 1.0000x reference)
import functools

import jax
import jax.numpy as jnp
from jax import lax
from jax.experimental import pallas as pl
from jax.experimental.pallas import tpu as pltpu
from jax.experimental.pallas import tpu_sc as plsc

D = 64
IDXW = 128
KSUB = 8
CHUNK = KSUB * IDXW
NC = 2
NS = 16
NW = NC * NS


def kernel(x, pe):
    B = x.size
    assert B % (NW * CHUNK) == 0
    n_chunks = B // (NW * CHUNK)
    idx2d = x.reshape(B // IDXW, IDXW)
    rows_per_w = (B // IDXW) // NW

    mesh = plsc.VectorSubcoreMesh(
        core_axis_name="c", subcore_axis_name="s", num_cores=NC, num_subcores=NS
    )

    @functools.partial(
        pl.kernel,
        mesh=mesh,
        out_type=jax.ShapeDtypeStruct((B, D), jnp.float32),
        scratch_types=[
            pltpu.VMEM((KSUB, IDXW), jnp.int32),
            pltpu.VMEM((CHUNK, D), jnp.float32),
            pltpu.SemaphoreType.DMA,
        ],
    )
    def gather_kernel(idx_hbm, table_hbm, out_hbm, idx_v, rows_v, sem):
        wid = lax.axis_index("s") * NC + lax.axis_index("c")
        row_base = wid * rows_per_w

        def body(c, carry):
            row0 = row_base + c * KSUB
            pltpu.sync_copy(idx_hbm.at[pl.ds(row0, KSUB)], idx_v)
            descs = []
            for j in range(KSUB):
                descs.append(
                    pltpu.async_copy(
                        table_hbm.at[idx_v.at[j]],
                        rows_v.at[pl.ds(j * IDXW, IDXW)],
                        sem,
                    )
                )
            for d in descs:
                d.wait()
            pltpu.sync_copy(rows_v, out_hbm.at[pl.ds(row0 * IDXW, CHUNK)])
            return carry

        lax.fori_loop(0, n_chunks, body, 0)

    out = gather_kernel(idx2d, pe)
    return out.reshape(x.shape + (D,))


# speedup vs baseline: 4.9827x; 4.9827x over previous
import functools

import jax
import jax.numpy as jnp
from jax import lax
from jax.experimental import pallas as pl
from jax.experimental.pallas import tpu as pltpu
from jax.experimental.pallas import tpu_sc as plsc

D = 64
IDXW = 128
KSUB = 8
CHUNK = KSUB * IDXW
NC = 2
NS = 16
NW = NC * NS


def kernel(x, pe):
    B = x.size
    assert B % (NW * CHUNK) == 0
    n_chunks = B // (NW * CHUNK)
    idx2d = x.reshape(B // IDXW, IDXW)
    rows_per_w = (B // IDXW) // NW

    mesh = plsc.VectorSubcoreMesh(
        core_axis_name="c", subcore_axis_name="s", num_cores=NC, num_subcores=NS
    )

    @functools.partial(
        pl.kernel,
        mesh=mesh,
        compiler_params=pltpu.CompilerParams(use_tc_tiling_on_sc=False),
        out_type=jax.ShapeDtypeStruct((B, D), jnp.float32),
        scratch_types=[
            pltpu.VMEM((KSUB, IDXW), jnp.int32),
            pltpu.VMEM((CHUNK, D), jnp.float32),
            pltpu.SemaphoreType.DMA,
        ],
    )
    def gather_kernel(idx_hbm, table_hbm, out_hbm, idx_v, rows_v, sem):
        wid = lax.axis_index("s") * NC + lax.axis_index("c")
        row_base = wid * rows_per_w

        def body(c, carry):
            row0 = row_base + c * KSUB
            pltpu.sync_copy(idx_hbm.at[pl.ds(row0, KSUB)], idx_v)
            descs = []
            for j in range(KSUB):
                descs.append(
                    pltpu.async_copy(
                        table_hbm.at[idx_v.at[j]],
                        rows_v.at[pl.ds(j * IDXW, IDXW)],
                        sem,
                    )
                )
            for d in descs:
                d.wait()
            pltpu.sync_copy(rows_v, out_hbm.at[pl.ds(row0 * IDXW, CHUNK)])
            return carry

        lax.fori_loop(0, n_chunks, body, 0)

    out = gather_kernel(idx2d, pe)
    return out.reshape(x.shape + (D,))


# speedup vs baseline: 5.1730x; 1.0382x over previous
import functools

import jax
import jax.numpy as jnp
from jax import lax
from jax.experimental import pallas as pl
from jax.experimental.pallas import tpu as pltpu
from jax.experimental.pallas import tpu_sc as plsc

D = 64
IDXW = 128
KSUB = 5
CHUNK = KSUB * IDXW
NBUF = 2
NC = 2
NS = 16
NW = NC * NS


def kernel(x, pe):
    B = x.size
    assert B % (NW * NBUF * CHUNK) == 0
    n_bodies = B // (NW * NBUF * CHUNK)
    idx2d = x.reshape(B // IDXW, IDXW)
    rows_per_w = (B // IDXW) // NW

    mesh = plsc.VectorSubcoreMesh(
        core_axis_name="c", subcore_axis_name="s", num_cores=NC, num_subcores=NS
    )

    @functools.partial(
        pl.kernel,
        mesh=mesh,
        compiler_params=pltpu.CompilerParams(use_tc_tiling_on_sc=False),
        out_type=jax.ShapeDtypeStruct((B, D), jnp.float32),
        scratch_types=[
            pltpu.VMEM((NBUF, KSUB, IDXW), jnp.int32),
            pltpu.VMEM((NBUF, CHUNK, D), jnp.float32),
            pltpu.SemaphoreType.DMA,
            pltpu.SemaphoreType.DMA,
            pltpu.SemaphoreType.DMA,
            pltpu.SemaphoreType.DMA,
        ],
    )
    def gather_kernel(idx_hbm, table_hbm, out_hbm, idx_v, rows_v,
                      gsem0, gsem1, osem0, osem1):
        wid = lax.axis_index("s") * NC + lax.axis_index("c")
        row_base = wid * rows_per_w
        gsems = (gsem0, gsem1)
        osems = (osem0, osem1)

        def fire_gathers(row0, b):
            pltpu.sync_copy(idx_hbm.at[pl.ds(row0, KSUB)], idx_v.at[b])
            return [
                pltpu.async_copy(
                    table_hbm.at[idx_v.at[b, j]],
                    rows_v.at[b, pl.ds(j * IDXW, IDXW)],
                    gsems[b],
                )
                for j in range(KSUB)
            ]

        def drain_out(b):
            pltpu.make_async_copy(
                rows_v.at[b], out_hbm.at[pl.ds(0, CHUNK)], osems[b]
            ).wait()

        def body(g, carry):
            r0 = row_base + g * (NBUF * KSUB)
            r1 = r0 + KSUB

            @pl.when(g > 0)
            def _():
                drain_out(0)

            d0 = fire_gathers(r0, 0)

            @pl.when(g > 0)
            def _():
                drain_out(1)

            d1 = fire_gathers(r1, 1)
            for d in d0:
                d.wait()
            pltpu.async_copy(
                rows_v.at[0], out_hbm.at[pl.ds(r0 * IDXW, CHUNK)], osems[0]
            )
            for d in d1:
                d.wait()
            pltpu.async_copy(
                rows_v.at[1], out_hbm.at[pl.ds(r1 * IDXW, CHUNK)], osems[1]
            )
            return carry

        lax.fori_loop(0, n_bodies, body, 0)
        drain_out(0)
        drain_out(1)

    out = gather_kernel(idx2d, pe)
    return out.reshape(x.shape + (D,))


# speedup vs baseline: 5.1743x; 1.0003x over previous
import functools

import jax
import jax.numpy as jnp
from jax import lax
from jax.experimental import pallas as pl
from jax.experimental.pallas import tpu as pltpu
from jax.experimental.pallas import tpu_sc as plsc

D = 64
CHUNK = 640
NBUF = 2
NC = 2
NS = 16
NW = NC * NS


def kernel(x, pe):
    B = x.size
    assert B % (NW * NBUF * CHUNK) == 0
    n_bodies = B // (NW * NBUF * CHUNK)
    xf = x.reshape(B)
    per_w = B // NW

    mesh = plsc.VectorSubcoreMesh(
        core_axis_name="c", subcore_axis_name="s", num_cores=NC, num_subcores=NS
    )

    @functools.partial(
        pl.kernel,
        mesh=mesh,
        compiler_params=pltpu.CompilerParams(use_tc_tiling_on_sc=False),
        out_type=jax.ShapeDtypeStruct((B, D), jnp.float32),
        scratch_types=[
            pltpu.VMEM((NBUF, CHUNK), jnp.int32),
            pltpu.VMEM((NBUF, CHUNK, D), jnp.float32),
            pltpu.SemaphoreType.DMA,
            pltpu.SemaphoreType.DMA,
            pltpu.SemaphoreType.DMA,
            pltpu.SemaphoreType.DMA,
        ],
    )
    def gather_kernel(idx_hbm, table_hbm, out_hbm, idx_v, rows_v,
                      gsem0, gsem1, osem0, osem1):
        wid = lax.axis_index("s") * NC + lax.axis_index("c")
        base = wid * per_w
        gsems = (gsem0, gsem1)
        osems = (osem0, osem1)

        def fire_gather(r0, b):
            pltpu.sync_copy(idx_hbm.at[pl.ds(r0, CHUNK)], idx_v.at[b])
            return pltpu.async_copy(
                table_hbm.at[idx_v.at[b]], rows_v.at[b], gsems[b]
            )

        def drain_out(b):
            pltpu.make_async_copy(
                rows_v.at[b], out_hbm.at[pl.ds(0, CHUNK)], osems[b]
            ).wait()

        def body(g, carry):
            r0 = base + g * (NBUF * CHUNK)
            r1 = r0 + CHUNK

            @pl.when(g > 0)
            def _():
                drain_out(0)

            d0 = fire_gather(r0, 0)

            @pl.when(g > 0)
            def _():
                drain_out(1)

            d1 = fire_gather(r1, 1)
            d0.wait()
            pltpu.async_copy(rows_v.at[0], out_hbm.at[pl.ds(r0, CHUNK)], osems[0])
            d1.wait()
            pltpu.async_copy(rows_v.at[1], out_hbm.at[pl.ds(r1, CHUNK)], osems[1])
            return carry

        lax.fori_loop(0, n_bodies, body, 0)
        drain_out(0)
        drain_out(1)

    out = gather_kernel(xf, pe)
    return out.reshape(x.shape + (D,))
